# baseline (speedup 1.0000x reference)
import functools
import math

import jax
import jax.numpy as jnp
from jax import lax
from jax.experimental import pallas as pl
from jax.experimental.pallas import tpu as pltpu
from jax.experimental.pallas import tpu_sc as plsc

D_MODEL = 64
SCALE = math.sqrt(D_MODEL)
LANES = 16
CHUNK = 128


def _make_sc_kernel(B, b_per_w, n_chunks, num_cores):
    mesh = plsc.VectorSubcoreMesh(core_axis_name="c", subcore_axis_name="s")

    @functools.partial(
        pl.kernel,
        mesh=mesh,
        compiler_params=pltpu.CompilerParams(use_tc_tiling_on_sc=False),
        out_type=jax.ShapeDtypeStruct((B, D_MODEL), jnp.float32),
        scratch_types=[
            pltpu.VMEM((CHUNK,), jnp.int32),
            pltpu.VMEM((CHUNK, D_MODEL), jnp.float32),
            pltpu.SemaphoreType.DMA,
        ],
    )
    def sc_gather(table_hbm, idx_hbm, out_hbm, idx_v, rows_v, sem):
        wid = lax.axis_index("s") * num_cores + lax.axis_index("c")
        base = wid * b_per_w

        def chunk_body(g, carry):
            off = base + g * CHUNK
            pltpu.sync_copy(idx_hbm.at[pl.ds(off, CHUNK)], idx_v)
            pltpu.async_copy(table_hbm.at[idx_v], rows_v, sem).wait()

            def scale_rows(r, c):
                for c4 in range(D_MODEL // LANES):
                    col = pl.ds(c4 * LANES, LANES)
                    rows_v[r, col] = rows_v[r, col] * SCALE
                return c

            lax.fori_loop(0, CHUNK, scale_rows, 0)
            pltpu.sync_copy(rows_v, out_hbm.at[pl.ds(off, CHUNK)])
            return carry

        lax.fori_loop(0, n_chunks, chunk_body, 0)

    return sc_gather


def kernel(src, W):
    info = plsc.get_sparse_core_info()
    nw = info.num_cores * info.num_subcores
    idx = src.reshape(-1).astype(jnp.int32)
    B = idx.shape[0]
    b_per_w = B // nw
    n_chunks = b_per_w // CHUNK
    sc_gather = _make_sc_kernel(B, b_per_w, n_chunks, info.num_cores)
    out = sc_gather(W, idx)
    return out.reshape(src.shape + (D_MODEL,))


# speedup vs baseline: 1.2797x; 1.2797x over previous
import functools
import math

import jax
import jax.numpy as jnp
from jax import lax
from jax.experimental import pallas as pl
from jax.experimental.pallas import tpu as pltpu
from jax.experimental.pallas import tpu_sc as plsc

D_MODEL = 64
SCALE = math.sqrt(D_MODEL)
LANES = 16
CHUNK = 128
NBUF = 4


def _make_sc_kernel(B, b_per_w, n_chunks, num_cores):
    mesh = plsc.VectorSubcoreMesh(core_axis_name="c", subcore_axis_name="s")
    n_groups = n_chunks // NBUF

    @functools.partial(
        pl.kernel,
        mesh=mesh,
        compiler_params=pltpu.CompilerParams(use_tc_tiling_on_sc=False),
        out_type=jax.ShapeDtypeStruct((B, D_MODEL), jnp.float32),
        scratch_types=[
            pltpu.VMEM((n_chunks, CHUNK), jnp.int32),
            *[pltpu.VMEM((CHUNK, D_MODEL), jnp.float32) for _ in range(2 * NBUF)],
            *[pltpu.SemaphoreType.DMA for _ in range(2 * NBUF)],
        ],
    )
    def sc_gather(table_hbm, idx_hbm, out_hbm, idx_all, *bufs_and_sems):
        inb = bufs_and_sems[:NBUF]
        outb = bufs_and_sems[NBUF:2 * NBUF]
        gsem = bufs_and_sems[2 * NBUF:3 * NBUF]
        osem = bufs_and_sems[3 * NBUF:]
        wid = lax.axis_index("s") * num_cores + lax.axis_index("c")
        base = wid * b_per_w

        pltpu.sync_copy(idx_hbm.at[wid], idx_all)
        for j in range(NBUF):
            pltpu.async_copy(table_hbm.at[idx_all.at[j]], inb[j], gsem[j])

        def group_body(g0, first=False, last=False):
            for j in range(NBUF):
                g = g0 * NBUF + j
                off = base + g * CHUNK
                pltpu.make_async_copy(
                    table_hbm.at[idx_all.at[0]], inb[j], gsem[j]).wait()
                if not first:
                    pltpu.make_async_copy(
                        outb[j], out_hbm.at[pl.ds(off, CHUNK)], osem[j]).wait()

                def scale_rows(r, c):
                    for rr in range(4):
                        for c4 in range(D_MODEL // LANES):
                            col = pl.ds(c4 * LANES, LANES)
                            outb[j][r * 4 + rr, col] = inb[j][r * 4 + rr, col] * SCALE
                    return c

                lax.fori_loop(0, CHUNK // 4, scale_rows, 0)
                pltpu.async_copy(outb[j], out_hbm.at[pl.ds(off, CHUNK)], osem[j])
                if not last:
                    pltpu.async_copy(
                        table_hbm.at[idx_all.at[g + NBUF]], inb[j], gsem[j])

        group_body(0, first=True)
        lax.fori_loop(1, n_groups - 1, lambda g0, c: (group_body(g0), c)[1], 0)
        group_body(n_groups - 1, last=True)
        for j in range(NBUF):
            off = base + ((n_groups - 1) * NBUF + j) * CHUNK
            pltpu.make_async_copy(
                outb[j], out_hbm.at[pl.ds(off, CHUNK)], osem[j]).wait()

    return sc_gather


def kernel(src, W):
    info = plsc.get_sparse_core_info()
    nw = info.num_cores * info.num_subcores
    idx = src.reshape(-1).astype(jnp.int32)
    B = idx.shape[0]
    b_per_w = B // nw
    n_chunks = b_per_w // CHUNK
    idx3 = idx.reshape(nw, n_chunks, CHUNK)
    sc_gather = _make_sc_kernel(B, b_per_w, n_chunks, info.num_cores)
    out = sc_gather(W, idx3)
    return out.reshape(src.shape + (D_MODEL,))
